# Initial kernel scaffold; baseline (speedup 1.0000x reference)
import jax
import jax.numpy as jnp
from jax.experimental import pallas as pl


def kernel(prediction, target):
    raise NotImplementedError("write your pallas kernel here")


# speedup vs baseline: 6.9234x; 6.9234x over previous
import functools
import math

import jax
import jax.numpy as jnp
from jax.experimental import pallas as pl
from jax.experimental.pallas import tpu as pltpu

_SMOOTH = 0.1
_VOCAB = 32000
_EPS = _SMOOTH / (_VOCAB - 1)
_CONF = 1.0 - _SMOOTH
_CD = _CONF - _EPS
_C = (_VOCAB - 1) * _EPS * math.log(_EPS) + _CONF * math.log(_CONF)


def _tc_body(col_block, pred_ref, tgt_ref, s_ref, n_ref):
    i = pl.program_id(0)
    t = tgt_ref[...]
    m = (t > 0).astype(jnp.float32)
    cols = jax.lax.broadcasted_iota(jnp.int32, pred_ref.shape, 1) + i * col_block
    eq = (cols == t).astype(jnp.float32)
    w = m * (_EPS + _CD * eq)
    part = jnp.sum(pred_ref[...] * w)

    @pl.when(i == 0)
    def _():
        s_ref[...] = jnp.zeros_like(s_ref)
        n_ref[...] = jnp.sum(m).reshape(1, 1)

    s_ref[...] += part.reshape(1, 1)


def kernel(prediction, target):
    rows = prediction.shape[0] * prediction.shape[1]
    vocab = prediction.shape[-1]
    pred = prediction.reshape(rows, vocab)
    tgt = target.reshape(rows, 1).astype(jnp.int32)

    col_block = 1280
    grid = (vocab // col_block,)
    s, n = pl.pallas_call(
        functools.partial(_tc_body, col_block),
        grid=grid,
        in_specs=[
            pl.BlockSpec((rows, col_block), lambda i: (0, i)),
            pl.BlockSpec((rows, 1), lambda i: (0, 0)),
        ],
        out_specs=[
            pl.BlockSpec((1, 1), lambda i: (0, 0)),
            pl.BlockSpec((1, 1), lambda i: (0, 0)),
        ],
        out_shape=[
            jax.ShapeDtypeStruct((1, 1), jnp.float32),
            jax.ShapeDtypeStruct((1, 1), jnp.float32),
        ],
    )(pred, tgt)
    nval = n[0, 0]
    return jnp.float32(_C) - s[0, 0] / nval
